# baseline (speedup 1.0000x reference)
import functools

import jax
import jax.numpy as jnp
from jax import lax
from jax.experimental import pallas as pl
from jax.experimental.pallas import tpu as pltpu
from jax.experimental.pallas import tpu_sc as plsc

NUM_CORES = 2
NUM_SUBCORES = 16
LANES = 16
NW = NUM_CORES * NUM_SUBCORES

BATCH = 16384
K = 64
D = 16
B_PER_W = BATCH // NW
CHUNK = 128
N_CHUNKS = B_PER_W // CHUNK
BLOCKS = CHUNK // LANES


def _project_body(f_ref, w_ref, b_ref, out_ref):
    out_ref[...] = lax.dot_general(
        f_ref[...], w_ref[...],
        dimension_numbers=(((1,), (1,)), ((), ())),
        preferred_element_type=jnp.float32,
    ) + b_ref[...]


def _project(F, W, b):
    return pl.pallas_call(
        _project_body,
        out_shape=jax.ShapeDtypeStruct((F.shape[0], W.shape[0]), jnp.float32),
    )(F, W, b.reshape(1, -1))


def _sc_body(users_hbm, items_hbm, gu_hbm, gi_hbm, tu_hbm, f_hbm, fp_hbm,
             xui_hbm, gamma_u_hbm, gamma_i_hbm, theta_u_hbm, effe_hbm,
             uidx_v, iidx_v, gu_v, gi_v, tu_v, fe_v, fp_v, xui_v, sem):
    wid = lax.axis_index("s") * NUM_CORES + lax.axis_index("c")
    base = wid * B_PER_W

    gu2, gi2, tu2, fp2 = gu_v, gi_v, tu_v, fp_v

    def chunk_body(c, carry):
        row0 = pl.multiple_of(base + c * CHUNK, CHUNK)
        pltpu.sync_copy(users_hbm.at[pl.ds(row0, CHUNK)], uidx_v)
        pltpu.sync_copy(items_hbm.at[pl.ds(row0, CHUNK)], iidx_v)
        cps = [
            pltpu.async_copy(gu_hbm.at[uidx_v], gu2, sem),
            pltpu.async_copy(gi_hbm.at[iidx_v], gi2, sem),
            pltpu.async_copy(tu_hbm.at[uidx_v], tu2, sem),
            pltpu.async_copy(f_hbm.at[iidx_v], fe_v, sem),
            pltpu.async_copy(fp_hbm.at[iidx_v], fp2, sem),
        ]
        for cp in cps:
            cp.wait()

        def block_body(bk, bcarry):
            lb = pl.multiple_of(bk * LANES, LANES)
            rows = lax.iota(jnp.int32, LANES) + lb
            acc = jnp.zeros((LANES,), jnp.float32)
            for k in range(K):
                col = jnp.full((LANES,), k, jnp.int32)
                acc = acc + (plsc.load_gather(gu_v, [rows, col]) *
                             plsc.load_gather(gi_v, [rows, col]))
            for dd in range(D):
                col = jnp.full((LANES,), dd, jnp.int32)
                acc = acc + (plsc.load_gather(tu_v, [rows, col]) *
                             plsc.load_gather(fp_v, [rows, col]))
            xui_v[pl.ds(lb, LANES)] = acc
            return bcarry

        lax.fori_loop(0, BLOCKS, block_body, 0)

        pltpu.sync_copy(gu2, gamma_u_hbm.at[pl.ds(row0, CHUNK), :])
        pltpu.sync_copy(gi2, gamma_i_hbm.at[pl.ds(row0, CHUNK), :])
        pltpu.sync_copy(tu2, theta_u_hbm.at[pl.ds(row0, CHUNK), :])
        pltpu.sync_copy(fe_v, effe_hbm.at[pl.ds(row0, CHUNK), :])
        pltpu.sync_copy(xui_v, xui_hbm.at[pl.ds(row0, CHUNK)])
        return carry

    lax.fori_loop(0, N_CHUNKS, chunk_body, 0)


@functools.partial(
    pl.kernel,
    out_type=(
        jax.ShapeDtypeStruct((BATCH,), jnp.float32),
        jax.ShapeDtypeStruct((BATCH, K), jnp.float32),
        jax.ShapeDtypeStruct((BATCH, K), jnp.float32),
        jax.ShapeDtypeStruct((BATCH, D), jnp.float32),
        jax.ShapeDtypeStruct((BATCH, D), jnp.float32),
    ),
    mesh=plsc.VectorSubcoreMesh(core_axis_name="c", subcore_axis_name="s"),
    compiler_params=pltpu.CompilerParams(
        needs_layout_passes=False, use_tc_tiling_on_sc=False),
    scratch_types=[
        pltpu.VMEM((CHUNK,), jnp.int32),
        pltpu.VMEM((CHUNK,), jnp.int32),
        pltpu.VMEM((CHUNK, K), jnp.float32),
        pltpu.VMEM((CHUNK, K), jnp.float32),
        pltpu.VMEM((CHUNK, D), jnp.float32),
        pltpu.VMEM((CHUNK, D), jnp.float32),
        pltpu.VMEM((CHUNK, D), jnp.float32),
        pltpu.VMEM((CHUNK,), jnp.float32),
        pltpu.SemaphoreType.DMA,
    ],
)
def _sc_kernel(*refs):
    _sc_body(*refs)


def kernel(users, items, Gu, Gi, Tu, F, W, b):
    fp = _project(F, W, b)
    xui, gamma_u, gamma_i, theta_u, effe_i = _sc_kernel(
        users[:, 0], items[:, 0], Gu, Gi, Tu, F, fp)
    return (xui, gamma_u, gamma_i, theta_u, effe_i)


# speedup vs baseline: 2.4444x; 2.4444x over previous
import functools

import jax
import jax.numpy as jnp
from jax import lax
from jax.experimental import pallas as pl
from jax.experimental.pallas import tpu as pltpu
from jax.experimental.pallas import tpu_sc as plsc

NUM_CORES = 2
NUM_SUBCORES = 16
LANES = 16
NW = NUM_CORES * NUM_SUBCORES

BATCH = 16384
K = 64
D = 16
PACK = 128
B_PER_W = BATCH // NW
CH = 64
N_CHUNKS = B_PER_W // CH
GROUPS = CH // LANES
NCH = BATCH // CH

IT_F = 0
IT_FP = D
IT_GI = 2 * D


def _project_body(f_ref, w_ref, b_ref, out_ref):
    out_ref[...] = lax.dot_general(
        f_ref[...], w_ref[...],
        dimension_numbers=(((1,), (1,)), ((), ())),
        preferred_element_type=jnp.float32,
    ) + b_ref[...]


def _project(F, W, b):
    return pl.pallas_call(
        _project_body,
        out_shape=jax.ShapeDtypeStruct((F.shape[0], W.shape[0]), jnp.float32),
    )(F, W, b.reshape(1, -1))


def _sc_body(su_hbm, si_hbm, ord_hbm, nf_hbm, gut_hbm, tut_hbm, it_hbm,
             guo_hbm, tuo_hbm, ito_hbm,
             su_v, si_v, ord_v, nf_v, gu_sl, tu_sl, it_v, guo_v, tuo_v,
             sem_it, sem_sl, sem_out):
    wid = lax.axis_index("s") * NUM_CORES + lax.axis_index("c")
    iot = lax.iota(jnp.int32, LANES)

    def chunk_body(c, carry):
        ch = wid * N_CHUNKS + c
        pltpu.sync_copy(su_hbm.at[ch], su_v)
        pltpu.sync_copy(si_hbm.at[ch], si_v)
        pltpu.sync_copy(ord_hbm.at[ch], ord_v)
        pltpu.sync_copy(nf_hbm.at[ch], nf_v)
        it_cp = pltpu.async_copy(it_hbm.at[si_v.at[0]], it_v, sem_it)

        for g in range(GROUPS):
            sl = pl.ds(g * LANES, LANES)
            su_vec = su_v[0, sl]
            nf_vec = nf_v[0, sl]
            for l in range(LANES):
                lg = g * LANES + l
                su_s = su_vec[l]

                @pl.when(nf_vec[l] != 0)
                def _fetch():
                    colbase = pl.multiple_of(
                        (lax.shift_right_logical(su_s, 7)) * PACK, PACK)
                    a = pltpu.async_copy(
                        gut_hbm.at[:, pl.ds(colbase, PACK)], gu_sl, sem_sl)
                    b = pltpu.async_copy(
                        tut_hbm.at[:, pl.ds(colbase, PACK)], tu_sl, sem_sl)
                    a.wait()
                    b.wait()

                colv = jnp.full((LANES,), su_s & (PACK - 1), jnp.int32)
                for q in range(K // LANES):
                    guo_v[lg, pl.ds(q * LANES, LANES)] = plsc.load_gather(
                        gu_sl, [iot + q * LANES, colv])
                tuo_v[lg, pl.ds(0, LANES)] = plsc.load_gather(
                    tu_sl, [iot, colv])

        it_cp.wait()

        for g in range(GROUPS):
            rows = iot + g * LANES
            acc = jnp.zeros((LANES,), jnp.float32)
            for k in range(K):
                acc = acc + (
                    plsc.load_gather(guo_v, [rows, jnp.full((LANES,), k, jnp.int32)])
                    * plsc.load_gather(
                        it_v, [rows, jnp.full((LANES,), IT_GI + k, jnp.int32)]))
            for dd in range(D):
                acc = acc + (
                    plsc.load_gather(tuo_v, [rows, jnp.full((LANES,), dd, jnp.int32)])
                    * plsc.load_gather(
                        it_v, [rows, jnp.full((LANES,), IT_FP + dd, jnp.int32)]))
            plsc.store_scatter(
                guo_v, [rows, jnp.full((LANES,), K, jnp.int32)], acc)

        pltpu.async_copy(guo_v, guo_hbm.at[ord_v.at[0]], sem_out).wait()
        pltpu.async_copy(tuo_v, tuo_hbm.at[ord_v.at[0]], sem_out).wait()
        pltpu.async_copy(it_v, ito_hbm.at[ord_v.at[0]], sem_out).wait()
        return carry

    lax.fori_loop(0, N_CHUNKS, chunk_body, 0)


@functools.partial(
    pl.kernel,
    out_type=(
        jax.ShapeDtypeStruct((BATCH, PACK), jnp.float32),
        jax.ShapeDtypeStruct((BATCH, PACK), jnp.float32),
        jax.ShapeDtypeStruct((BATCH, PACK), jnp.float32),
    ),
    mesh=plsc.VectorSubcoreMesh(core_axis_name="c", subcore_axis_name="s"),
    compiler_params=pltpu.CompilerParams(
        needs_layout_passes=False, use_tc_tiling_on_sc=True),
    scratch_types=[
        pltpu.VMEM((1, CH), jnp.int32),
        pltpu.VMEM((1, CH), jnp.int32),
        pltpu.VMEM((1, CH), jnp.int32),
        pltpu.VMEM((1, CH), jnp.int32),
        pltpu.VMEM((K, PACK), jnp.float32),
        pltpu.VMEM((D, PACK), jnp.float32),
        pltpu.VMEM((CH, PACK), jnp.float32),
        pltpu.VMEM((CH, PACK), jnp.float32),
        pltpu.VMEM((CH, PACK), jnp.float32),
        pltpu.SemaphoreType.DMA,
        pltpu.SemaphoreType.DMA,
        pltpu.SemaphoreType.DMA,
    ],
)
def _sc_kernel(*refs):
    _sc_body(*refs)


def kernel(users, items, Gu, Gi, Tu, F, W, b):
    u = users[:, 0]
    it = items[:, 0]
    fp = _project(F, W, b)
    itab = jnp.pad(jnp.concatenate([F, fp, Gi], axis=1),
                   ((0, 0), (0, PACK - 2 * D - K)))
    order = jnp.argsort(u).astype(jnp.int32)
    su = jnp.take(u, order)
    si = jnp.take(it, order)
    slab = lax.shift_right_logical(su, 7)
    pos = lax.iota(jnp.int32, BATCH)
    nf = jnp.where(
        (pos % B_PER_W == 0) | (pos == 0)
        | (slab != jnp.roll(slab, 1)), 1, 0).astype(jnp.int32)
    shp = (NCH, 1, CH)
    guo, tuo, ito = _sc_kernel(
        su.reshape(shp), si.reshape(shp), order.reshape(shp), nf.reshape(shp),
        Gu.T, Tu.T, itab)
    xui = guo[:, K]
    gamma_u = guo[:, :K]
    gamma_i = ito[:, IT_GI:IT_GI + K]
    theta_u = tuo[:, :D]
    effe_i = ito[:, IT_F:IT_F + D]
    return (xui, gamma_u, gamma_i, theta_u, effe_i)


# speedup vs baseline: 3.0134x; 1.2328x over previous
import functools

import jax
import jax.numpy as jnp
from jax import lax
from jax.experimental import pallas as pl
from jax.experimental.pallas import tpu as pltpu
from jax.experimental.pallas import tpu_sc as plsc

NUM_CORES = 2
NUM_SUBCORES = 16
LANES = 16
NW = NUM_CORES * NUM_SUBCORES

BATCH = 16384
K = 64
D = 16
PACK = 128
B_PER_W = BATCH // NW
CH = 64
N_CHUNKS = B_PER_W // CH
GROUPS = CH // LANES
NCH = BATCH // CH
PF = 8

IT_F = 0
IT_FP = D
IT_GI = 2 * D
OUT_TU = K
OUT_XUI = K + D


def _project_body(f_ref, w_ref, b_ref, out_ref):
    out_ref[...] = lax.dot_general(
        f_ref[...], w_ref[...],
        dimension_numbers=(((1,), (1,)), ((), ())),
        preferred_element_type=jnp.float32,
    ) + b_ref[...]


def _project(F, W, b):
    return pl.pallas_call(
        _project_body,
        out_shape=jax.ShapeDtypeStruct((F.shape[0], W.shape[0]), jnp.float32),
    )(F, W, b.reshape(1, -1))


def _sc_body(su_hbm, si_hbm, ord_hbm, nf_hbm, rw_hbm, pf_hbm, pro_hbm,
             gut_hbm, tut_hbm, it_hbm,
             guo_hbm, ito_hbm,
             su_v, si_v, ord_v, nf_v, rw_v, pf_v, pro_v,
             gu_ar, tu_ar, it_v, guo_v,
             sem_it, sem_out, sem_sl):
    wid = lax.axis_index("s") * NUM_CORES + lax.axis_index("c")
    iot = lax.iota(jnp.int32, LANES)

    def slab_fetch(slab_id, slot):
        colb = pl.multiple_of(slab_id * PACK, PACK)
        gslot = pl.multiple_of(slot * K, K)
        tslot = pl.multiple_of(slot * D, D)
        pltpu.async_copy(gut_hbm.at[:, pl.ds(colb, PACK)],
                         gu_ar.at[pl.ds(gslot, K), :], sem_sl.at[slot])
        pltpu.async_copy(tut_hbm.at[:, pl.ds(colb, PACK)],
                         tu_ar.at[pl.ds(tslot, D), :], sem_sl.at[slot])

    def slab_drain(slot):
        gslot = pl.multiple_of(slot * K, K)
        tslot = pl.multiple_of(slot * D, D)
        pltpu.make_async_copy(gut_hbm.at[:, pl.ds(0, PACK)],
                              gu_ar.at[pl.ds(gslot, K), :],
                              sem_sl.at[slot]).wait()
        pltpu.make_async_copy(tut_hbm.at[:, pl.ds(0, PACK)],
                              tu_ar.at[pl.ds(tslot, D), :],
                              sem_sl.at[slot]).wait()

    pltpu.sync_copy(pro_hbm.at[wid], pro_v)
    provec = pro_v[0, pl.ds(0, LANES)]
    for i in range(PF - 1):
        slab_fetch(provec[i], i)

    def chunk_body(c, rw_last):
        ch = wid * N_CHUNKS + c
        pltpu.sync_copy(su_hbm.at[ch], su_v)
        pltpu.sync_copy(si_hbm.at[ch], si_v)
        pltpu.sync_copy(ord_hbm.at[ch], ord_v)
        pltpu.sync_copy(nf_hbm.at[ch], nf_v)
        pltpu.sync_copy(rw_hbm.at[ch], rw_v)
        pltpu.sync_copy(pf_hbm.at[ch], pf_v)
        it_cp = pltpu.async_copy(it_hbm.at[si_v.at[0]], it_v, sem_it)

        for g in range(GROUPS):
            sl = pl.ds(g * LANES, LANES)
            su_vec = su_v[0, sl]
            nf_vec = nf_v[0, sl]
            rw_vec = rw_v[0, sl]
            pf_vec = pf_v[0, sl]
            for l in range(LANES):
                lg = g * LANES + l
                su_s = su_vec[l]
                w_slot = rw_vec[l]

                @pl.when(nf_vec[l] != 0)
                def _rotate():
                    slab_drain(w_slot)
                    slab_fetch(pf_vec[l], (w_slot + PF - 1) & (PF - 1))

                colv = jnp.full((LANES,), su_s & (PACK - 1), jnp.int32)
                gbase = w_slot * K
                for q in range(K // LANES):
                    guo_v[lg, pl.ds(q * LANES, LANES)] = plsc.load_gather(
                        gu_ar, [gbase + iot + q * LANES, colv])
                guo_v[lg, pl.ds(OUT_TU, D)] = plsc.load_gather(
                    tu_ar, [w_slot * D + iot, colv])

        it_cp.wait()

        for g in range(GROUPS):
            rows = iot + g * LANES
            acc = jnp.zeros((LANES,), jnp.float32)
            for k in range(K):
                acc = acc + (
                    plsc.load_gather(
                        guo_v, [rows, jnp.full((LANES,), k, jnp.int32)])
                    * plsc.load_gather(
                        it_v, [rows, jnp.full((LANES,), IT_GI + k, jnp.int32)]))
            for dd in range(D):
                acc = acc + (
                    plsc.load_gather(
                        guo_v, [rows, jnp.full((LANES,), OUT_TU + dd, jnp.int32)])
                    * plsc.load_gather(
                        it_v, [rows, jnp.full((LANES,), IT_FP + dd, jnp.int32)]))
            plsc.store_scatter(
                guo_v, [rows, jnp.full((LANES,), OUT_XUI, jnp.int32)], acc)

        pltpu.async_copy(guo_v, guo_hbm.at[ord_v.at[0]], sem_out).wait()
        pltpu.async_copy(it_v, ito_hbm.at[ord_v.at[0]], sem_out).wait()
        return rw_v[0, pl.ds(CH - LANES, LANES)][LANES - 1]

    rw_last = lax.fori_loop(0, N_CHUNKS, chunk_body, jnp.int32(0))

    for i in range(PF - 1):
        slab_drain((rw_last + 1 + i) & (PF - 1))


@functools.partial(
    pl.kernel,
    out_type=(
        jax.ShapeDtypeStruct((BATCH, PACK), jnp.float32),
        jax.ShapeDtypeStruct((BATCH, PACK), jnp.float32),
    ),
    mesh=plsc.VectorSubcoreMesh(core_axis_name="c", subcore_axis_name="s"),
    compiler_params=pltpu.CompilerParams(
        needs_layout_passes=False, use_tc_tiling_on_sc=True),
    scratch_types=[
        pltpu.VMEM((1, CH), jnp.int32),
        pltpu.VMEM((1, CH), jnp.int32),
        pltpu.VMEM((1, CH), jnp.int32),
        pltpu.VMEM((1, CH), jnp.int32),
        pltpu.VMEM((1, CH), jnp.int32),
        pltpu.VMEM((1, CH), jnp.int32),
        pltpu.VMEM((1, PACK), jnp.int32),
        pltpu.VMEM((PF * K, PACK), jnp.float32),
        pltpu.VMEM((PF * D, PACK), jnp.float32),
        pltpu.VMEM((CH, PACK), jnp.float32),
        pltpu.VMEM((CH, PACK), jnp.float32),
        pltpu.SemaphoreType.DMA,
        pltpu.SemaphoreType.DMA,
        pltpu.SemaphoreType.DMA((PF,)),
    ],
)
def _sc_kernel(*refs):
    _sc_body(*refs)


def kernel(users, items, Gu, Gi, Tu, F, W, b):
    u = users[:, 0]
    it = items[:, 0]
    fp = _project(F, W, b)
    itab = jnp.pad(jnp.concatenate([F, fp, Gi], axis=1),
                   ((0, 0), (0, PACK - 2 * D - K)))

    order = jnp.argsort(u).astype(jnp.int32)
    su = jnp.take(u, order)
    si = jnp.take(it, order)
    slab = lax.shift_right_logical(su, 7)
    pos = lax.iota(jnp.int32, BATCH)
    nf = jnp.where((pos % B_PER_W == 0) | (slab != jnp.roll(slab, 1)),
                   1, 0).astype(jnp.int32)
    runid = jnp.cumsum(nf) - 1
    sor = jnp.zeros((BATCH,), jnp.int32).at[runid].set(slab)
    pfs = jnp.take(sor, jnp.clip(runid + PF - 1, 0, BATCH - 1))
    runid0 = jnp.take(runid, (pos // B_PER_W) * B_PER_W)
    rw = (runid - runid0) & (PF - 1)
    pro = jnp.take(sor, runid[::B_PER_W][:, None]
                   + jnp.arange(PF - 1, dtype=jnp.int32)[None, :])
    pro3 = jnp.zeros((NW, 1, PACK), jnp.int32).at[:, 0, :PF - 1].set(pro)

    shp = (NCH, 1, CH)
    guo, ito = _sc_kernel(
        su.reshape(shp), si.reshape(shp), order.reshape(shp),
        nf.reshape(shp), rw.reshape(shp), pfs.reshape(shp), pro3,
        Gu.T, Tu.T, itab)
    xui = guo[:, OUT_XUI]
    gamma_u = guo[:, :K]
    gamma_i = ito[:, IT_GI:IT_GI + K]
    theta_u = guo[:, OUT_TU:OUT_TU + D]
    effe_i = ito[:, IT_F:IT_F + D]
    return (xui, gamma_u, gamma_i, theta_u, effe_i)
